# baseline (speedup 1.0000x reference)
import functools

import jax
import jax.numpy as jnp
from jax import lax
from jax.experimental import pallas as pl
from jax.experimental.pallas import tpu as pltpu
from jax.experimental.pallas import tpu_sc as plsc

_BATCH = 16384
_DIM = 2048
_CHUNK_ROWS = 32


@functools.lru_cache(maxsize=1)
def _make_sc_copy():
    info = plsc.get_sparse_core_info()
    nw = info.num_cores * info.num_subcores
    nc = info.num_cores
    rows_per_tile = _BATCH // nw
    nchunks = rows_per_tile // _CHUNK_ROWS
    mesh = plsc.VectorSubcoreMesh(core_axis_name="c", subcore_axis_name="s")

    @functools.partial(
        pl.kernel,
        mesh=mesh,
        out_type=jax.ShapeDtypeStruct((_BATCH, _DIM), jnp.float32),
        scratch_types=[
            pltpu.VMEM_SHARED((16, 1, _CHUNK_ROWS, _DIM), jnp.float32),
            pltpu.VMEM((1, _CHUNK_ROWS, _DIM), jnp.float32),
            pltpu.SemaphoreType.DMA((2,)),
            pltpu.SemaphoreType.DMA((2,)),
        ],
    )
    def sc_copy(h_hbm, out_hbm, shared, tbuf, rsem, wsem):
        sid = lax.axis_index("s")
        wid = sid * nc + lax.axis_index("c")
        base = wid * rows_per_tile
        bufs = (shared.at[sid, 0], tbuf.at[0])
        depth = 2

        def rd(i, b):
            return pltpu.make_async_copy(
                h_hbm.at[pl.ds(base + i * _CHUNK_ROWS, _CHUNK_ROWS), :],
                bufs[b], rsem.at[b])

        def wr(i, b):
            return pltpu.make_async_copy(
                bufs[b],
                out_hbm.at[pl.ds(base + i * _CHUNK_ROWS, _CHUNK_ROWS), :],
                wsem.at[b])

        for j in range(depth - 1):
            rd(j, j).start()
        for i in range(nchunks):
            b = i % depth
            if i >= 1:
                wr(i - 1, (i - 1) % depth).wait()
            if i + depth - 1 < nchunks:
                rd(i + depth - 1, (i + depth - 1) % depth).start()
            rd(i, b).wait()
            wr(i, b).start()
        wr(nchunks - 1, (nchunks - 1) % depth).wait()

    return sc_copy


def kernel(h):
    return _make_sc_copy()(h)


# speedup vs baseline: 1.3147x; 1.3147x over previous
import jax
import jax.numpy as jnp
from jax.experimental import pallas as pl

_BATCH = 16384
_DIM = 2048
_BLOCK_ROWS = 1024


def _copy_body(h_ref, o_ref):
    o_ref[...] = h_ref[...]


def kernel(h):
    grid = (_BATCH // _BLOCK_ROWS,)
    return pl.pallas_call(
        _copy_body,
        grid=grid,
        in_specs=[pl.BlockSpec((_BLOCK_ROWS, _DIM), lambda i: (i, 0))],
        out_specs=pl.BlockSpec((_BLOCK_ROWS, _DIM), lambda i: (i, 0)),
        out_shape=jax.ShapeDtypeStruct((_BATCH, _DIM), jnp.float32),
    )(h)
